# baseline (speedup 1.0000x reference)
import jax
import jax.numpy as jnp
from jax.experimental import pallas as pl

B = 4096
S = 200
D = 64
NSPAN = S // 2
ROWS = B * NSPAN
BR = 8192
BL = 512


def _split_body(x_ref, s_ref, e_ref):
    x = x_ref[...]
    s_ref[...] = x[:, :D]
    e_ref[...] = x[:, D:]


def _len_body(l_ref):
    l_ref[...] = jnp.full((BL, NSPAN), 2, jnp.int32)


@jax.jit
def kernel(input):
    x = input.reshape(ROWS, 2 * D)
    s, e = pl.pallas_call(
        _split_body,
        grid=(ROWS // BR,),
        in_specs=[pl.BlockSpec((BR, 2 * D), lambda i: (i, 0))],
        out_specs=(
            pl.BlockSpec((BR, D), lambda i: (i, 0)),
            pl.BlockSpec((BR, D), lambda i: (i, 0)),
        ),
        out_shape=(
            jax.ShapeDtypeStruct((ROWS, D), jnp.float32),
            jax.ShapeDtypeStruct((ROWS, D), jnp.float32),
        ),
    )(x)
    length = pl.pallas_call(
        _len_body,
        grid=(B // BL,),
        out_specs=pl.BlockSpec((BL, NSPAN), lambda i: (i, 0)),
        out_shape=jax.ShapeDtypeStruct((B, NSPAN), jnp.int32),
    )()
    return (s.reshape(B, NSPAN, D), e.reshape(B, NSPAN, D), length)


# speedup vs baseline: 1.7386x; 1.7386x over previous
import jax
import jax.numpy as jnp
from jax.experimental import pallas as pl

B = 4096
S = 200
D = 64
NSPAN = S // 2
BT = 128


def _body(x_ref, s_ref, e_ref, l_ref):
    x = x_ref[...]
    s_ref[...] = x[:, :, :D]
    e_ref[...] = x[:, :, D:]
    l_ref[...] = jnp.full((BT, NSPAN), 2, jnp.int32)


@jax.jit
def kernel(input):
    x = input.reshape(B, NSPAN, 2 * D)
    return pl.pallas_call(
        _body,
        grid=(B // BT,),
        in_specs=[pl.BlockSpec((BT, NSPAN, 2 * D), lambda i: (i, 0, 0))],
        out_specs=(
            pl.BlockSpec((BT, NSPAN, D), lambda i: (i, 0, 0)),
            pl.BlockSpec((BT, NSPAN, D), lambda i: (i, 0, 0)),
            pl.BlockSpec((BT, NSPAN), lambda i: (i, 0)),
        ),
        out_shape=(
            jax.ShapeDtypeStruct((B, NSPAN, D), jnp.float32),
            jax.ShapeDtypeStruct((B, NSPAN, D), jnp.float32),
            jax.ShapeDtypeStruct((B, NSPAN), jnp.int32),
        ),
    )(x)
